# baseline (speedup 1.0000x reference)
import functools
import jax
import jax.numpy as jnp
from jax import lax
from jax.experimental import pallas as pl
from jax.experimental.pallas import tpu as pltpu, tpu_sc as plsc

B, S, D = 4, 8192, 1024
TOT = S * D
NC, NS, L = 2, 16, 16
NW = NC * NS
PER_W = TOT // NW
CH = 8192
N_CHUNKS = PER_W // CH
UNROLL = 8


def _sc_body(in_hbm, tab_hbm, out_hbm,
             tab0, tab1, in0, in1, in2, in3, o0, o1, o2, o3,
             st0, st1, si0, si1, si2, si3, so0, so1, so2, so3):
    tabs, stabs = [tab0, tab1], [st0, st1]
    ins, sins = [in0, in1, in2, in3], [si0, si1, si2, si3]
    outs, souts = [o0, o1, o2, o3], [so0, so1, so2, so3]

    wid = lax.axis_index("s") * NC + lax.axis_index("c")
    base = wid * PER_W

    def tab_copy(c, p):
        pltpu.async_copy(tab_hbm.at[pl.ds(base + c * CH, CH)], tabs[p], stabs[p])

    def tab_wait(c, p):
        pltpu.make_async_copy(
            tab_hbm.at[pl.ds(base + c * CH, CH)], tabs[p], stabs[p]).wait()

    def in_copy(c, b):
        pltpu.async_copy(in_hbm.at[b, pl.ds(base + c * CH, CH)], ins[b], sins[b])

    def in_wait(c, b):
        pltpu.make_async_copy(
            in_hbm.at[b, pl.ds(base + c * CH, CH)], ins[b], sins[b]).wait()

    def out_start(c, b):
        pltpu.async_copy(outs[b], out_hbm.at[b, pl.ds(base + c * CH, CH)], souts[b])

    def out_wait(c, b):
        pltpu.make_async_copy(
            outs[b], out_hbm.at[b, pl.ds(base + c * CH, CH)], souts[b]).wait()

    tab_copy(0, 0)
    for b in range(B):
        in_copy(0, b)

    def super_step(c2, _):
        for phase in (0, 1):
            c = 2 * c2 + phase
            if phase == 0:
                tab_copy(c + 1, 1)
            else:
                pl.when(c2 < N_CHUNKS // 2 - 1)(lambda: tab_copy(c + 1, 0))
            tab_wait(c, phase)
            for b in range(B):
                in_wait(c, b)
                if phase == 0:
                    pl.when(c2 > 0)(lambda: out_wait(c - 1, b))
                else:
                    out_wait(c - 1, b)

                in_b, tab_p, out_b = ins[b], tabs[phase], outs[b]

                @plsc.parallel_loop(0, CH, step=L, unroll=UNROLL)
                def _(i):
                    ds = pl.ds(i, L)
                    x = in_b[ds]
                    out_b[ds] = jnp.where(x == 0.0, x, tab_p[ds])

                out_start(c, b)
                if phase == 0:
                    in_copy(c + 1, b)
                else:
                    pl.when(c2 < N_CHUNKS // 2 - 1)(lambda: in_copy(c + 1, b))
        return 0

    lax.fori_loop(0, N_CHUNKS // 2, super_step, 0)
    for b in range(B):
        out_wait(N_CHUNKS - 1, b)


def _kernel_sc(inputs, pos_table):
    x = inputs.reshape(B, TOT)
    t = pos_table.reshape(TOT)
    mesh = plsc.VectorSubcoreMesh(core_axis_name="c", subcore_axis_name="s")
    f = pl.kernel(
        _sc_body,
        mesh=mesh,
        out_type=jax.ShapeDtypeStruct((B, TOT), jnp.float32),
        scratch_types=(
            [pltpu.VMEM((CH,), jnp.float32) for _ in range(10)]
            + [pltpu.SemaphoreType.DMA for _ in range(10)]
        ),
    )
    return f(x, t).reshape(B, S, D)


def _tc_body(x_ref, t_ref, o_ref):
    x = x_ref[...]
    t = t_ref[...]
    o_ref[...] = jnp.where(x == 0.0, x, t[None, :, :])


def _kernel_tc(inputs, pos_table):
    b, s, d = inputs.shape
    s_blk = min(512, s)
    return pl.pallas_call(
        _tc_body,
        grid=(s // s_blk,),
        in_specs=[
            pl.BlockSpec((b, s_blk, d), lambda i: (0, i, 0)),
            pl.BlockSpec((s_blk, d), lambda i: (i, 0)),
        ],
        out_specs=pl.BlockSpec((b, s_blk, d), lambda i: (0, i, 0)),
        out_shape=jax.ShapeDtypeStruct((b, s, d), inputs.dtype),
        compiler_params=pltpu.CompilerParams(
            dimension_semantics=("arbitrary",)),
    )(inputs, pos_table)


def kernel(inputs, pos_table):
    return _kernel_tc(inputs, pos_table)


# speedup vs baseline: 1.0028x; 1.0028x over previous
import jax
import jax.numpy as jnp
from jax import lax
from jax.experimental import pallas as pl
from jax.experimental.pallas import tpu as pltpu, tpu_sc as plsc

B, S, D = 4, 8192, 1024
TOT = S * D
NC, NS, L = 2, 16, 16
NW = NC * NS
PER_W = TOT // NW
CH = 8192
N_CHUNKS = PER_W // CH
UNROLL = 8


def _sc_body(in_hbm, tab_hbm, out_hbm,
             tab0, tab1, in0, in1, in2, in3, o0, o1, o2, o3,
             st0, st1, si0, si1, si2, si3, so0, so1, so2, so3):
    tabs, stabs = [tab0, tab1], [st0, st1]
    ins, sins = [in0, in1, in2, in3], [si0, si1, si2, si3]
    outs, souts = [o0, o1, o2, o3], [so0, so1, so2, so3]

    wid = lax.axis_index("s") * NC + lax.axis_index("c")
    base = wid * PER_W

    def tab_copy(c, p):
        pltpu.async_copy(tab_hbm.at[pl.ds(base + c * CH, CH)], tabs[p], stabs[p])

    def tab_wait(c, p):
        pltpu.make_async_copy(
            tab_hbm.at[pl.ds(base + c * CH, CH)], tabs[p], stabs[p]).wait()

    def in_copy(c, b):
        pltpu.async_copy(in_hbm.at[b, pl.ds(base + c * CH, CH)], ins[b], sins[b])

    def in_wait(c, b):
        pltpu.make_async_copy(
            in_hbm.at[b, pl.ds(base + c * CH, CH)], ins[b], sins[b]).wait()

    def out_start(c, b):
        pltpu.async_copy(outs[b], out_hbm.at[b, pl.ds(base + c * CH, CH)], souts[b])

    def out_wait(c, b):
        pltpu.make_async_copy(
            outs[b], out_hbm.at[b, pl.ds(base + c * CH, CH)], souts[b]).wait()

    tab_copy(0, 0)
    for b in range(B):
        in_copy(0, b)

    def super_step(c2, _):
        for phase in (0, 1):
            c = 2 * c2 + phase
            if phase == 0:
                tab_copy(c + 1, 1)
            else:
                pl.when(c2 < N_CHUNKS // 2 - 1)(lambda: tab_copy(c + 1, 0))
            tab_wait(c, phase)
            for b in range(B):
                in_wait(c, b)
                if phase == 0:
                    pl.when(c2 > 0)(lambda: out_wait(c - 1, b))
                else:
                    out_wait(c - 1, b)

                in_b, tab_p, out_b = ins[b], tabs[phase], outs[b]

                @plsc.parallel_loop(0, CH, step=L, unroll=UNROLL)
                def _(i):
                    ds = pl.ds(i, L)
                    x = in_b[ds]
                    out_b[ds] = jnp.where(x == 0.0, x, tab_p[ds])

                out_start(c, b)
                if phase == 0:
                    in_copy(c + 1, b)
                else:
                    pl.when(c2 < N_CHUNKS // 2 - 1)(lambda: in_copy(c + 1, b))
        return 0

    lax.fori_loop(0, N_CHUNKS // 2, super_step, 0)
    for b in range(B):
        out_wait(N_CHUNKS - 1, b)


def _kernel_sc(inputs, pos_table):
    x = inputs.reshape(B, TOT)
    t = pos_table.reshape(TOT)
    mesh = plsc.VectorSubcoreMesh(core_axis_name="c", subcore_axis_name="s")
    f = pl.kernel(
        _sc_body,
        mesh=mesh,
        out_type=jax.ShapeDtypeStruct((B, TOT), jnp.float32),
        scratch_types=(
            [pltpu.VMEM((CH,), jnp.float32) for _ in range(10)]
            + [pltpu.SemaphoreType.DMA for _ in range(10)]
        ),
    )
    return f(x, t).reshape(B, S, D)


def _tc_body(x_ref, t_ref, o_ref):
    x = x_ref[...]
    t = t_ref[...]
    o_ref[...] = jnp.where(x == 0.0, x, t[None, :, :])


def _kernel_tc(inputs, pos_table):
    b, s, d = inputs.shape
    s_blk = min(512, s)
    return pl.pallas_call(
        _tc_body,
        grid=(s // s_blk,),
        in_specs=[
            pl.BlockSpec((b, s_blk, d), lambda i: (0, i, 0)),
            pl.BlockSpec((s_blk, d), lambda i: (i, 0)),
        ],
        out_specs=pl.BlockSpec((b, s_blk, d), lambda i: (0, i, 0)),
        out_shape=jax.ShapeDtypeStruct((b, s, d), inputs.dtype),
    )(inputs, pos_table)


def kernel(inputs, pos_table):
    return _kernel_tc(inputs, pos_table)
